# Initial kernel scaffold; baseline (speedup 1.0000x reference)
import jax
import jax.numpy as jnp
from jax.experimental import pallas as pl


def kernel(hidden_states, router_w, expert_gate_w, expert_up_w, expert_down_w, shared_gate_w, shared_up_w, shared_down_w, shared_expert_gate_w):
    raise NotImplementedError("write your pallas kernel here")


# speedup vs baseline: 1.1245x; 1.1245x over previous
import jax
import jax.numpy as jnp
from jax.experimental import pallas as pl
from jax.experimental.pallas import tpu as pltpu

_B, _S, _D, _E, _K, _F, _FS = 64, 1, 2048, 64, 8, 512, 512
_T = _B * _S
_FB = 256
_NF = _F // _FB


def _dot_t(a, b, precision=None):
    return jax.lax.dot_general(
        a, b, (((1,), (1,)), ((), ())),
        preferred_element_type=jnp.float32, precision=precision)


def _moe_kernel(x_ref, rw_ref, gw_ref, uw_ref, dw_ref,
                sgw_ref, suw_ref, sdw_ref, segw_ref,
                out_ref, logits_ref, w_ref):
    e = pl.program_id(0)
    fi = pl.program_id(1)
    x = x_ref[...]

    @pl.when(jnp.logical_and(e == 0, fi == 0))
    def _prologue():
        logits = _dot_t(x, rw_ref[...])
        logits_ref[...] = logits
        m = jnp.max(logits, axis=1, keepdims=True)
        ex = jnp.exp(logits - m)
        probs = ex / jnp.sum(ex, axis=1, keepdims=True)
        col = jax.lax.broadcasted_iota(jnp.int32, (_T, _E), 1)
        remaining = probs
        acc = jnp.zeros_like(probs)
        for _ in range(_K):
            mx = jnp.max(remaining, axis=1, keepdims=True)
            is_max = remaining == mx
            first = jnp.min(jnp.where(is_max, col, _E), axis=1, keepdims=True)
            pick = col == first
            acc = jnp.where(pick, probs, acc)
            remaining = jnp.where(pick, -jnp.inf, remaining)
        w_ref[...] = acc / jnp.sum(acc, axis=1, keepdims=True)
        sg = _dot_t(x, sgw_ref[...])
        su = _dot_t(x, suw_ref[...])
        sh = (sg * jax.nn.sigmoid(sg)) * su
        sd = jax.lax.dot_general(
            sh, sdw_ref[...], (((1,), (1,)), ((), ())),
            preferred_element_type=jnp.float32)
        tok_gate = jax.nn.sigmoid(_dot_t(x, segw_ref[...]))
        out_ref[...] = tok_gate * sd

    w_e = jnp.sum(
        jnp.where(jax.lax.broadcasted_iota(jnp.int32, (_T, _E), 1) == e,
                  w_ref[...], 0.0),
        axis=1, keepdims=True)
    g = _dot_t(x, gw_ref[0])
    u = _dot_t(x, uw_ref[0])
    h = (g * jax.nn.sigmoid(g)) * u * w_e
    contrib = jax.lax.dot_general(
        h, dw_ref[0], (((1,), (1,)), ((), ())),
        preferred_element_type=jnp.float32)
    out_ref[...] += contrib


@jax.jit
def kernel(hidden_states, router_w, expert_gate_w, expert_up_w, expert_down_w,
           shared_gate_w, shared_up_w, shared_down_w, shared_expert_gate_w):
    x = hidden_states.reshape(_T, _D)
    out, logits = pl.pallas_call(
        _moe_kernel,
        grid=(_E, _NF),
        in_specs=[
            pl.BlockSpec((_T, _D), lambda e, f: (0, 0)),
            pl.BlockSpec((_E, _D), lambda e, f: (0, 0)),
            pl.BlockSpec((1, _FB, _D), lambda e, f: (e, f, 0)),
            pl.BlockSpec((1, _FB, _D), lambda e, f: (e, f, 0)),
            pl.BlockSpec((1, _D, _FB), lambda e, f: (e, 0, f)),
            pl.BlockSpec((_FS, _D), lambda e, f: (0, 0)),
            pl.BlockSpec((_FS, _D), lambda e, f: (0, 0)),
            pl.BlockSpec((_D, _FS), lambda e, f: (0, 0)),
            pl.BlockSpec((1, _D), lambda e, f: (0, 0)),
        ],
        out_specs=[
            pl.BlockSpec((_T, _D), lambda e, f: (0, 0)),
            pl.BlockSpec((_T, _E), lambda e, f: (0, 0)),
        ],
        out_shape=[
            jax.ShapeDtypeStruct((_T, _D), jnp.float32),
            jax.ShapeDtypeStruct((_T, _E), jnp.float32),
        ],
        scratch_shapes=[pltpu.VMEM((_T, _E), jnp.float32)],
        compiler_params=pltpu.CompilerParams(
            dimension_semantics=("arbitrary", "arbitrary")),
    )(x, router_w, expert_gate_w, expert_up_w, expert_down_w,
      shared_gate_w, shared_up_w, shared_down_w, shared_expert_gate_w)
    return out.reshape(_B, _S, _D), logits


# speedup vs baseline: 1.2905x; 1.1477x over previous
import jax
import jax.numpy as jnp
from jax.experimental import pallas as pl
from jax.experimental.pallas import tpu as pltpu

_B, _S, _D, _E, _K, _F, _FS = 64, 1, 2048, 64, 8, 512, 512
_T = _B * _S
_FB = 512
_NF = _F // _FB


def _dot_t(a, b, precision=None):
    return jax.lax.dot_general(
        a, b, (((1,), (1,)), ((), ())),
        preferred_element_type=jnp.float32, precision=precision)


def _dot_t_bf16(a, b):
    return jax.lax.dot_general(
        a.astype(jnp.bfloat16), b.astype(jnp.bfloat16),
        (((1,), (1,)), ((), ())), preferred_element_type=jnp.float32)


def _moe_kernel(x_ref, rw_ref, gw_ref, uw_ref, dw_ref,
                sgw_ref, suw_ref, sdw_ref, segw_ref,
                out_ref, logits_ref, w_ref):
    e = pl.program_id(0)
    fi = pl.program_id(1)
    x = x_ref[...]

    @pl.when(jnp.logical_and(e == 0, fi == 0))
    def _prologue():
        logits = _dot_t(x, rw_ref[...])
        logits_ref[...] = logits
        m = jnp.max(logits, axis=1, keepdims=True)
        ex = jnp.exp(logits - m)
        probs = ex / jnp.sum(ex, axis=1, keepdims=True)
        col = jax.lax.broadcasted_iota(jnp.int32, (_T, _E), 1)
        remaining = probs
        acc = jnp.zeros_like(probs)
        for _ in range(_K):
            mx = jnp.max(remaining, axis=1, keepdims=True)
            is_max = remaining == mx
            first = jnp.min(jnp.where(is_max, col, _E), axis=1, keepdims=True)
            pick = col == first
            acc = jnp.where(pick, probs, acc)
            remaining = jnp.where(pick, -jnp.inf, remaining)
        w_ref[...] = acc / jnp.sum(acc, axis=1, keepdims=True)
        sg = _dot_t_bf16(x, sgw_ref[...])
        su = _dot_t_bf16(x, suw_ref[...])
        sh = (sg * jax.nn.sigmoid(sg)) * su
        sd = _dot_t_bf16(sh, sdw_ref[...])
        tok_gate = jax.nn.sigmoid(_dot_t(x, segw_ref[...]))
        out_ref[...] = tok_gate * sd

    w_e = jnp.sum(
        jnp.where(jax.lax.broadcasted_iota(jnp.int32, (_T, _E), 1) == e,
                  w_ref[...], 0.0),
        axis=1, keepdims=True)
    g = _dot_t_bf16(x, gw_ref[0])
    u = _dot_t_bf16(x, uw_ref[0])
    h = (g * jax.nn.sigmoid(g)) * u * w_e
    contrib = _dot_t_bf16(h, dw_ref[0])
    out_ref[...] += contrib


@jax.jit
def kernel(hidden_states, router_w, expert_gate_w, expert_up_w, expert_down_w,
           shared_gate_w, shared_up_w, shared_down_w, shared_expert_gate_w):
    x = hidden_states.reshape(_T, _D)
    out, logits = pl.pallas_call(
        _moe_kernel,
        grid=(_E, _NF),
        in_specs=[
            pl.BlockSpec((_T, _D), lambda e, f: (0, 0)),
            pl.BlockSpec((_E, _D), lambda e, f: (0, 0)),
            pl.BlockSpec((1, _FB, _D), lambda e, f: (e, f, 0)),
            pl.BlockSpec((1, _FB, _D), lambda e, f: (e, f, 0)),
            pl.BlockSpec((1, _D, _FB), lambda e, f: (e, 0, f)),
            pl.BlockSpec((_FS, _D), lambda e, f: (0, 0)),
            pl.BlockSpec((_FS, _D), lambda e, f: (0, 0)),
            pl.BlockSpec((_D, _FS), lambda e, f: (0, 0)),
            pl.BlockSpec((1, _D), lambda e, f: (0, 0)),
        ],
        out_specs=[
            pl.BlockSpec((_T, _D), lambda e, f: (0, 0)),
            pl.BlockSpec((_T, _E), lambda e, f: (0, 0)),
        ],
        out_shape=[
            jax.ShapeDtypeStruct((_T, _D), jnp.float32),
            jax.ShapeDtypeStruct((_T, _E), jnp.float32),
        ],
        scratch_shapes=[pltpu.VMEM((_T, _E), jnp.float32)],
        compiler_params=pltpu.CompilerParams(
            dimension_semantics=("arbitrary", "arbitrary")),
    )(x, router_w, expert_gate_w, expert_up_w, expert_down_w,
      shared_gate_w, shared_up_w, shared_down_w, shared_expert_gate_w)
    return out.reshape(_B, _S, _D), logits
